# baseline (speedup 1.0000x reference)
import functools

import jax
import jax.numpy as jnp
from jax import lax
from jax.experimental import pallas as pl
from jax.experimental.pallas import tpu as pltpu
from jax.experimental.pallas import tpu_sc as plsc

L = 16
NW = 32
IDX_CHUNK = 128
FACTOR = 64


def _bpr_body(b_per_w, n_chunks,
              user_ref, item_i_ref, item_j_ref, eu_ref, ei_ref,
              out_i_ref, out_j_ref,
              idx_u, idx_i, idx_j, rows_u, rows_i, rows_j,
              obuf_i, obuf_j, sem):
  wid = lax.axis_index("s") * 2 + lax.axis_index("c")
  base = wid * b_per_w

  for c in range(n_chunks):
    pltpu.sync_copy(user_ref.at[pl.ds(base + c * IDX_CHUNK, IDX_CHUNK)],
                    idx_u.at[c])
    pltpu.sync_copy(item_i_ref.at[pl.ds(base + c * IDX_CHUNK, IDX_CHUNK)],
                    idx_i.at[c])
    pltpu.sync_copy(item_j_ref.at[pl.ds(base + c * IDX_CHUNK, IDX_CHUNK)],
                    idx_j.at[c])

  copies = []
  for c in range(n_chunks):
    dst = pl.ds(c * IDX_CHUNK, IDX_CHUNK)
    copies.append(pltpu.async_copy(eu_ref.at[idx_u.at[c]], rows_u.at[dst], sem))
    copies.append(pltpu.async_copy(ei_ref.at[idx_i.at[c]], rows_i.at[dst], sem))
    copies.append(pltpu.async_copy(ei_ref.at[idx_j.at[c]], rows_j.at[dst], sem))
  for cp in copies:
    cp.wait()

  lane = lax.iota(jnp.int32, L)

  def chunk(ci, carry):
    ridx = ci * L + lane
    acc_i = jnp.zeros((L,), jnp.float32)
    acc_j = jnp.zeros((L,), jnp.float32)
    for d in range(FACTOR):
      cidx = jnp.full((L,), d, jnp.int32)
      gu = plsc.load_gather(rows_u, [ridx, cidx])
      gi = plsc.load_gather(rows_i, [ridx, cidx])
      gj = plsc.load_gather(rows_j, [ridx, cidx])
      acc_i = acc_i + gu * gi
      acc_j = acc_j + gu * gj
    b0 = ci * L
    obuf_i[pl.ds(b0, L)] = acc_i
    obuf_j[pl.ds(b0, L)] = acc_j
    return carry

  lax.fori_loop(0, b_per_w // L, chunk, 0)

  pltpu.sync_copy(obuf_i, out_i_ref.at[pl.ds(base, b_per_w)])
  pltpu.sync_copy(obuf_j, out_j_ref.at[pl.ds(base, b_per_w)])


def kernel(user, item_i, item_j, embed_user, embed_item):
  batch = user.shape[0]
  assert batch % (NW * IDX_CHUNK) == 0
  b_per_w = batch // NW
  n_chunks = b_per_w // IDX_CHUNK
  factor = embed_user.shape[1]
  assert factor == FACTOR

  mesh = plsc.VectorSubcoreMesh(core_axis_name="c", subcore_axis_name="s",
                                num_cores=2, num_subcores=16)
  out_sds = jax.ShapeDtypeStruct((batch,), jnp.float32)
  run = pl.kernel(
      functools.partial(_bpr_body, b_per_w, n_chunks),
      out_type=(out_sds, out_sds),
      mesh=mesh,
      scratch_types=[
          pltpu.VMEM((n_chunks, IDX_CHUNK), jnp.int32),
          pltpu.VMEM((n_chunks, IDX_CHUNK), jnp.int32),
          pltpu.VMEM((n_chunks, IDX_CHUNK), jnp.int32),
          pltpu.VMEM((b_per_w, FACTOR), jnp.float32),
          pltpu.VMEM((b_per_w, FACTOR), jnp.float32),
          pltpu.VMEM((b_per_w, FACTOR), jnp.float32),
          pltpu.VMEM((b_per_w,), jnp.float32),
          pltpu.VMEM((b_per_w,), jnp.float32),
          pltpu.SemaphoreType.DMA,
      ],
      compiler_params=pltpu.CompilerParams(needs_layout_passes=False,
                                           use_tc_tiling_on_sc=False),
  )
  return run(user.astype(jnp.int32), item_i.astype(jnp.int32),
             item_j.astype(jnp.int32), embed_user, embed_item)
